# baseline (speedup 1.0000x reference)
import functools
import math

import jax
import jax.numpy as jnp
from jax import lax
from jax.experimental import pallas as pl
from jax.experimental.pallas import tpu as pltpu
from jax.experimental.pallas import tpu_sc as plsc

_NUM_WORKERS = 32
_CHUNK = 128
_LANES = 16


def _gather_scale_body(n_chunks, d, scale, idx_hbm, table_hbm, out_hbm,
                       idx_v, rows_v, sem):
  c = lax.axis_index("c")
  s = lax.axis_index("s")
  wid = s * 2 + c
  per_w = n_chunks * _CHUNK

  pltpu.sync_copy(idx_hbm.at[wid], idx_v)

  def chunk_body(g, carry):
    pltpu.async_copy(table_hbm.at[idx_v.at[g]], rows_v, sem).wait()

    def scale_body(i, carry2):
      for j in range(d // _LANES):
        sl = (i, pl.ds(j * _LANES, _LANES))
        rows_v[sl] = rows_v[sl] * scale
      return carry2

    lax.fori_loop(0, _CHUNK, scale_body, 0, unroll=4)

    pltpu.sync_copy(rows_v, out_hbm.at[pl.ds(wid * per_w + g * _CHUNK, _CHUNK)])
    return carry

  lax.fori_loop(0, n_chunks, chunk_body, 0)


def kernel(x, table):
  b, s = x.shape
  v, d = table.shape
  total = b * s
  assert total % (_NUM_WORKERS * _CHUNK) == 0
  assert d % _LANES == 0
  n_chunks = total // (_NUM_WORKERS * _CHUNK)
  scale = math.sqrt(d)

  idx = x.reshape(_NUM_WORKERS, n_chunks, _CHUNK).astype(jnp.int32)

  mesh = plsc.VectorSubcoreMesh(core_axis_name="c", subcore_axis_name="s")
  body = functools.partial(_gather_scale_body, n_chunks, d, scale)
  out = pl.kernel(
      body,
      mesh=mesh,
      out_type=jax.ShapeDtypeStruct((total, d), jnp.float32),
      compiler_params=pltpu.CompilerParams(use_tc_tiling_on_sc=False),
      scratch_types=[
          pltpu.VMEM((n_chunks, _CHUNK), jnp.int32),
          pltpu.VMEM((_CHUNK, d), jnp.float32),
          pltpu.SemaphoreType.DMA,
      ],
  )(idx, table)

  return out.reshape(b, s, d)


# speedup vs baseline: 1.1599x; 1.1599x over previous
import functools
import math

import jax
import jax.numpy as jnp
from jax import lax
from jax.experimental import pallas as pl
from jax.experimental.pallas import tpu as pltpu
from jax.experimental.pallas import tpu_sc as plsc

_NUM_WORKERS = 32
_CHUNK = 512
_LANES = 16


def _gather_scale_body(n_chunks, d, scale, idx_hbm, table_hbm, out_hbm,
                       idx_v, rows_v, gsem0, gsem1, osem0, osem1):
  c = lax.axis_index("c")
  s = lax.axis_index("s")
  wid = s * 2 + c
  per_w = n_chunks * _CHUNK
  out_base = wid * per_w

  pltpu.sync_copy(idx_hbm.at[wid], idx_v)

  bufs = (rows_v.at[0], rows_v.at[1])
  gsems = (gsem0, gsem1)
  osems = (osem0, osem1)

  def start_gather(g, k):
    pltpu.async_copy(table_hbm.at[idx_v.at[g]], bufs[k], gsems[k])

  def wait_gather(g, k):
    pltpu.make_async_copy(table_hbm.at[idx_v.at[g]], bufs[k], gsems[k]).wait()

  def out_ref(g, k):
    return out_hbm.at[pl.ds(out_base + g * _CHUNK, _CHUNK)]

  def start_out(g, k):
    pltpu.async_copy(bufs[k], out_ref(g, k), osems[k])

  def wait_out(g, k):
    pltpu.make_async_copy(bufs[k], out_ref(g, k), osems[k]).wait()

  def scale_chunk(k):
    buf = bufs[k]

    def scale_body(i, carry):
      for j in range(d // _LANES):
        sl = (i, pl.ds(j * _LANES, _LANES))
        buf[sl] = buf[sl] * scale
      return carry

    lax.fori_loop(0, _CHUNK, scale_body, 0, unroll=8)

  start_gather(0, 0)

  @pl.loop(0, n_chunks, step=2)
  def _(gg):
    for k in range(2):
      g = gg + k
      nk = 1 - k
      if k == 0:
        @pl.when(gg > 0)
        def _():
          wait_out(g - 1, nk)

        start_gather(g + 1, nk)
      else:
        @pl.when(g + 1 < n_chunks)
        def _():
          wait_out(g - 1, nk)
          start_gather(g + 1, nk)

      wait_gather(g, k)
      scale_chunk(k)
      start_out(g, k)

  wait_out(n_chunks - 2, 0)
  wait_out(n_chunks - 1, 1)


def kernel(x, table):
  b, s = x.shape
  v, d = table.shape
  total = b * s
  assert total % (_NUM_WORKERS * _CHUNK) == 0
  assert d % _LANES == 0
  n_chunks = total // (_NUM_WORKERS * _CHUNK)
  assert n_chunks % 2 == 0
  scale = math.sqrt(d)

  idx = x.reshape(_NUM_WORKERS, n_chunks, _CHUNK).astype(jnp.int32)

  mesh = plsc.VectorSubcoreMesh(core_axis_name="c", subcore_axis_name="s")
  body = functools.partial(_gather_scale_body, n_chunks, d, scale)
  out = pl.kernel(
      body,
      mesh=mesh,
      out_type=jax.ShapeDtypeStruct((total, d), jnp.float32),
      compiler_params=pltpu.CompilerParams(use_tc_tiling_on_sc=False),
      scratch_types=[
          pltpu.VMEM((n_chunks, _CHUNK), jnp.int32),
          pltpu.VMEM((2, _CHUNK, d), jnp.float32),
          pltpu.SemaphoreType.DMA,
          pltpu.SemaphoreType.DMA,
          pltpu.SemaphoreType.DMA,
          pltpu.SemaphoreType.DMA,
      ],
  )(idx, table)

  return out.reshape(b, s, d)
